# baseline (speedup 1.0000x reference)
import jax
import jax.numpy as jnp
from jax.experimental import pallas as pl
from jax.experimental.pallas import tpu as pltpu

_BM = 200


def _dot_nt(x, w):
    return jax.lax.dot_general(x, w, (((1,), (1,)), ((), ())),
                               preferred_element_type=jnp.float32)


def _gcn_kernel(a_ref, ax_ref, w0_ref, b0_ref, a0_ref, w1_ref, b1_ref,
                a1_ref, out_ref, t_ref):
    @pl.when(pl.program_id(0) == 0)
    def _compute_t():
        x = _dot_nt(ax_ref[...], w0_ref[...]) + b0_ref[...]
        a0 = a0_ref[0]
        x = jnp.where(x >= 0, x, a0 * x)
        t_ref[...] = _dot_nt(x, w1_ref[...]) + b1_ref[...]

    acc = jnp.dot(a_ref[...], t_ref[...], preferred_element_type=jnp.float32)
    a1 = a1_ref[0]
    out_ref[...] = jnp.where(acc >= 0, acc, a1 * acc)


def kernel(A, AX, W0, b0, a0, W1, b1, a1):
    n, d = AX.shape
    h = W0.shape[0]

    out = pl.pallas_call(
        _gcn_kernel,
        grid=(n // _BM,),
        in_specs=[
            pl.BlockSpec((_BM, n), lambda i: (i, 0)),
            pl.BlockSpec((n, d), lambda i: (0, 0)),
            pl.BlockSpec((h, d), lambda i: (0, 0)),
            pl.BlockSpec((1, h), lambda i: (0, 0)),
            pl.BlockSpec(memory_space=pltpu.SMEM),
            pl.BlockSpec((h, h), lambda i: (0, 0)),
            pl.BlockSpec((1, h), lambda i: (0, 0)),
            pl.BlockSpec(memory_space=pltpu.SMEM),
        ],
        out_specs=pl.BlockSpec((_BM, h), lambda i: (i, 0)),
        out_shape=jax.ShapeDtypeStruct((n, h), jnp.float32),
        scratch_shapes=[pltpu.VMEM((n, h), jnp.float32)],
        compiler_params=pltpu.CompilerParams(
            dimension_semantics=("arbitrary",),
        ),
    )(A, AX, W0, b0.reshape(1, h), a0.reshape(1), W1, b1.reshape(1, h),
      a1.reshape(1))

    return out[None, :, :]


# speedup vs baseline: 1.0069x; 1.0069x over previous
import jax
import jax.numpy as jnp
from jax.experimental import pallas as pl
from jax.experimental.pallas import tpu as pltpu

_BM = 400


def _dot_nt(x, w):
    return jax.lax.dot_general(x, w, (((1,), (1,)), ((), ())),
                               preferred_element_type=jnp.float32)


def _gcn_kernel(a_ref, ax_ref, w0_ref, b0_ref, a0_ref, w1_ref, b1_ref,
                a1_ref, out_ref, t_ref):
    @pl.when(pl.program_id(0) == 0)
    def _compute_t():
        x = _dot_nt(ax_ref[...], w0_ref[...]) + b0_ref[...]
        a0 = a0_ref[0]
        x = jnp.where(x >= 0, x, a0 * x)
        t_ref[...] = _dot_nt(x, w1_ref[...]) + b1_ref[...]

    acc = jnp.dot(a_ref[...], t_ref[...], preferred_element_type=jnp.float32)
    a1 = a1_ref[0]
    out_ref[...] = jnp.where(acc >= 0, acc, a1 * acc)


def kernel(A, AX, W0, b0, a0, W1, b1, a1):
    n, d = AX.shape
    h = W0.shape[0]

    out = pl.pallas_call(
        _gcn_kernel,
        grid=(n // _BM,),
        in_specs=[
            pl.BlockSpec((_BM, n), lambda i: (i, 0)),
            pl.BlockSpec((n, d), lambda i: (0, 0)),
            pl.BlockSpec((h, d), lambda i: (0, 0)),
            pl.BlockSpec((1, h), lambda i: (0, 0)),
            pl.BlockSpec(memory_space=pltpu.SMEM),
            pl.BlockSpec((h, h), lambda i: (0, 0)),
            pl.BlockSpec((1, h), lambda i: (0, 0)),
            pl.BlockSpec(memory_space=pltpu.SMEM),
        ],
        out_specs=pl.BlockSpec((_BM, h), lambda i: (i, 0)),
        out_shape=jax.ShapeDtypeStruct((n, h), jnp.float32),
        scratch_shapes=[pltpu.VMEM((n, h), jnp.float32)],
        compiler_params=pltpu.CompilerParams(
            dimension_semantics=("arbitrary",),
        ),
    )(A, AX, W0, b0.reshape(1, h), a0.reshape(1), W1, b1.reshape(1, h),
      a1.reshape(1))

    return out[None, :, :]
